# baseline (speedup 1.0000x reference)
import functools

import jax
import jax.numpy as jnp
from jax.experimental import pallas as pl
from jax.experimental.pallas import tpu as pltpu


def _round_up(v, m):
    return ((v + m - 1) // m) * m


def _make_body(T, H, W, Cin, hid, kh, kw, Cpad, TR):
    ph, pw = kh // 2, kw // 2
    NR = H // TR
    f32 = jnp.float32
    bf16 = jnp.bfloat16

    def body(x_ref, hc0_ref, w_ref, b_ref, out_ref, c_ref):
        hc0 = hc0_ref[0].astype(f32)
        c_ref[...] = hc0[..., hid:2 * hid]
        x_ref[0, 0, ph:ph + H, pw:pw + W, 0:hid] = hc0[..., :hid].astype(bf16)
        b = b_ref[0].astype(f32)

        def conv_tile(t, r0):
            pieces = [x_ref[0, t, pl.ds(r0 + ki, TR), kj:kj + W, :]
                      for ki in range(kh) for kj in range(kw)]
            patches = jnp.concatenate(pieces, axis=-1)
            acc = jax.lax.dot_general(
                patches, w_ref[...],
                dimension_numbers=(((2,), (0,)), ((), ())),
                preferred_element_type=f32)
            return acc + b

        def gate_math(acc, r0):
            th = jnp.tanh(acc)
            lane = jax.lax.broadcasted_iota(jnp.int32, acc.shape, 2)
            act = jnp.where(lane < 3 * hid, 0.5 * th + 0.5, th)
            i = act[..., 0 * hid:1 * hid]
            f = act[..., 1 * hid:2 * hid]
            o = act[..., 2 * hid:3 * hid]
            g = act[..., 3 * hid:4 * hid]
            c_cur = c_ref[pl.ds(r0, TR), :, :]
            c_next = f * c_cur + i * g
            c_ref[pl.ds(r0, TR), :, :] = c_next
            return o * jnp.tanh(c_next)

        def step(t, carry):
            def row_tile(r, cc):
                r0 = pl.multiple_of(r * TR, TR)
                h_next = gate_math(conv_tile(t, r0), r0)
                x_ref[0, t + 1, pl.ds(r0 + ph, TR), pw:pw + W, 0:hid] = (
                    h_next.astype(bf16))
                return cc
            jax.lax.fori_loop(0, NR, row_tile, 0, unroll=True)
            return carry

        jax.lax.fori_loop(0, T - 1, step, 0)

        def row_tile_last(r, cc):
            r0 = pl.multiple_of(r * TR, TR)
            h_next = gate_math(conv_tile(T - 1, r0), r0)
            out_ref[0, pl.ds(r0, TR), :, 0:hid] = h_next.astype(out_ref.dtype)
            return cc
        jax.lax.fori_loop(0, NR, row_tile_last, 0, unroll=True)
        out_ref[0, :, :, hid:2 * hid] = c_ref[...].astype(out_ref.dtype)

    return body


@functools.partial(jax.jit, static_argnames=("input_dim", "hidden_dim",
                                             "kernel_size"))
def _convlstm_seq(xs, hc0, w_packed, b_packed, *,
                  input_dim, hidden_dim, kernel_size):
    B, T, H, W, Cin = xs.shape
    hid = hidden_dim
    kh, kw = kernel_size
    C = Cin + hid
    Cpad = _round_up(C, 64)
    K = kh * kw * Cpad
    assert Cin == input_dim
    assert hc0.shape == (B, H, W, 2 * hid)
    assert w_packed.shape == (K, 4 * hid)

    TR = next((tr for tr in (8, 4, 2, 1) if H % tr == 0), 1)

    ph, pw = kh // 2, kw // 2
    Hp, Wp = H + 2 * ph, W + 2 * pw

    x_slab = jnp.pad(xs.astype(jnp.bfloat16),
                     ((0, 0), (0, 0), (ph, ph), (pw, pw),
                      (hid, Cpad - hid - Cin)))

    gate_scale = jnp.concatenate([jnp.full((3 * hid,), 0.5, jnp.float32),
                                  jnp.ones((hid,), jnp.float32)])
    w_s = (w_packed * gate_scale).astype(jnp.bfloat16)
    b_s = b_packed * gate_scale

    body = _make_body(T, H, W, Cin, hid, kh, kw, Cpad, TR)
    return pl.pallas_call(
        body,
        out_shape=jax.ShapeDtypeStruct((B, H, W, 2 * hid), xs.dtype),
        grid_spec=pltpu.PrefetchScalarGridSpec(
            num_scalar_prefetch=0,
            grid=(B,),
            in_specs=[
                pl.BlockSpec((1, T, Hp, Wp, Cpad),
                             lambda b: (b, 0, 0, 0, 0)),
                pl.BlockSpec((1, H, W, 2 * hid), lambda b: (b, 0, 0, 0)),
                pl.BlockSpec((K, 4 * hid), lambda b: (0, 0)),
                pl.BlockSpec((1, 4 * hid), lambda b: (0, 0)),
            ],
            out_specs=pl.BlockSpec((1, H, W, 2 * hid), lambda b: (b, 0, 0, 0)),
            scratch_shapes=[
                pltpu.VMEM((H, W, hid), jnp.float32),
            ]),
        compiler_params=pltpu.CompilerParams(
            dimension_semantics=("parallel",)),
    )(x_slab, hc0, w_s, b_s)


def kernel(xs, hc0, w_packed, b_packed):
    return _convlstm_seq(xs, hc0, w_packed, b_packed,
                         input_dim=64, hidden_dim=64, kernel_size=(3, 3))


# speedup vs baseline: 1.2259x; 1.2259x over previous
import functools

import jax
import jax.numpy as jnp
from jax.experimental import pallas as pl
from jax.experimental.pallas import tpu as pltpu


def _round_up(v, m):
    return ((v + m - 1) // m) * m


def _make_body(T, H, W, Cin, hid, kh, kw, Cpad, TR):
    ph, pw = kh // 2, kw // 2
    NR = H // TR
    f32 = jnp.float32
    bf16 = jnp.bfloat16

    def body(x_ref, hc0_ref, w_ref, b_ref, out_ref, slab_ref, c_ref):
        slab_ref[...] = jnp.zeros_like(slab_ref)
        hc0 = hc0_ref[0].astype(f32)
        c_ref[...] = hc0[..., hid:2 * hid]
        slab_ref[0, ph:ph + H, pw:pw + W, 0:hid] = hc0[..., :hid].astype(bf16)
        slab_ref[0, ph:ph + H, pw:pw + W, hid:hid + Cin] = (
            x_ref[0, 0].astype(bf16))
        b = b_ref[0].astype(f32)

        def conv_tile(t, r0):
            pieces = [slab_ref[t, pl.ds(r0 + ki, TR), kj:kj + W, :]
                      for ki in range(kh) for kj in range(kw)]
            patches = jnp.concatenate(pieces, axis=-1)
            acc = jax.lax.dot_general(
                patches, w_ref[...],
                dimension_numbers=(((2,), (0,)), ((), ())),
                preferred_element_type=f32)
            return acc + b

        def gate_math(acc, r0):
            th = jnp.tanh(acc)
            lane = jax.lax.broadcasted_iota(jnp.int32, acc.shape, 2)
            act = jnp.where(lane < 3 * hid, 0.5 * th + 0.5, th)
            i = act[..., 0 * hid:1 * hid]
            f = act[..., 1 * hid:2 * hid]
            o = act[..., 2 * hid:3 * hid]
            g = act[..., 3 * hid:4 * hid]
            c_cur = c_ref[pl.ds(r0, TR), :, :]
            c_next = f * c_cur + i * g
            c_ref[pl.ds(r0, TR), :, :] = c_next
            return o * jnp.tanh(c_next)

        def step(t, carry):
            slab_ref[t + 1, ph:ph + H, pw:pw + W, hid:hid + Cin] = (
                x_ref[0, t + 1].astype(bf16))

            def row_tile(r, cc):
                r0 = pl.multiple_of(r * TR, TR)
                h_next = gate_math(conv_tile(t, r0), r0)
                slab_ref[t + 1, pl.ds(r0 + ph, TR), pw:pw + W, 0:hid] = (
                    h_next.astype(bf16))
                return cc
            jax.lax.fori_loop(0, NR, row_tile, 0, unroll=True)
            return carry

        jax.lax.fori_loop(0, T - 1, step, 0)

        def row_tile_last(r, cc):
            r0 = pl.multiple_of(r * TR, TR)
            h_next = gate_math(conv_tile(T - 1, r0), r0)
            out_ref[0, pl.ds(r0, TR), :, 0:hid] = h_next.astype(out_ref.dtype)
            return cc
        jax.lax.fori_loop(0, NR, row_tile_last, 0, unroll=True)
        out_ref[0, :, :, hid:2 * hid] = c_ref[...].astype(out_ref.dtype)

    return body


@functools.partial(jax.jit, static_argnames=("input_dim", "hidden_dim",
                                             "kernel_size"))
def _convlstm_seq(xs, hc0, w_packed, b_packed, *,
                  input_dim, hidden_dim, kernel_size):
    B, T, H, W, Cin = xs.shape
    hid = hidden_dim
    kh, kw = kernel_size
    C = Cin + hid
    Cpad = _round_up(C, 64)
    K = kh * kw * Cpad
    assert Cin == input_dim
    assert hc0.shape == (B, H, W, 2 * hid)
    assert w_packed.shape == (K, 4 * hid)

    TR = next((tr for tr in (8, 4, 2, 1) if H % tr == 0), 1)

    ph, pw = kh // 2, kw // 2
    Hp, Wp = H + 2 * ph, W + 2 * pw

    gate_scale = jnp.concatenate([jnp.full((3 * hid,), 0.5, jnp.float32),
                                  jnp.ones((hid,), jnp.float32)])
    w_s = (w_packed * gate_scale).astype(jnp.bfloat16)
    b_s = b_packed * gate_scale

    body = _make_body(T, H, W, Cin, hid, kh, kw, Cpad, TR)
    return pl.pallas_call(
        body,
        out_shape=jax.ShapeDtypeStruct((B, H, W, 2 * hid), xs.dtype),
        grid_spec=pltpu.PrefetchScalarGridSpec(
            num_scalar_prefetch=0,
            grid=(B,),
            in_specs=[
                pl.BlockSpec((1, T, H, W, Cin), lambda b: (b, 0, 0, 0, 0)),
                pl.BlockSpec((1, H, W, 2 * hid), lambda b: (b, 0, 0, 0)),
                pl.BlockSpec((K, 4 * hid), lambda b: (0, 0)),
                pl.BlockSpec((1, 4 * hid), lambda b: (0, 0)),
            ],
            out_specs=pl.BlockSpec((1, H, W, 2 * hid), lambda b: (b, 0, 0, 0)),
            scratch_shapes=[
                pltpu.VMEM((T, Hp, Wp, Cpad), jnp.bfloat16),
                pltpu.VMEM((H, W, hid), jnp.float32),
            ]),
        compiler_params=pltpu.CompilerParams(
            dimension_semantics=("parallel",)),
    )(xs, hc0, w_s, b_s)


def kernel(xs, hc0, w_packed, b_packed):
    return _convlstm_seq(xs, hc0, w_packed, b_packed,
                         input_dim=64, hidden_dim=64, kernel_size=(3, 3))


# speedup vs baseline: 1.4191x; 1.1576x over previous
import functools

import jax
import jax.numpy as jnp
from jax.experimental import pallas as pl
from jax.experimental.pallas import tpu as pltpu


def _round_up(v, m):
    return ((v + m - 1) // m) * m


def _make_body(T, H, W, Cin, hid, kh, kw, Cpad, TR, NB):
    ph, pw = kh // 2, kw // 2
    NR = H // TR
    f32 = jnp.float32
    bf16 = jnp.bfloat16

    def body(x_ref, hc0_ref, w_ref, b_ref, out_ref, slab_ref, c_ref):
        slab_ref[...] = jnp.zeros_like(slab_ref)
        for j in range(NB):
            hc0 = hc0_ref[j].astype(f32)
            c_ref[j] = hc0[..., hid:2 * hid]
            slab_ref[j, 0, ph:ph + H, pw:pw + W, 0:hid] = (
                hc0[..., :hid].astype(bf16))
            slab_ref[j, 0, ph:ph + H, pw:pw + W, hid:hid + Cin] = (
                x_ref[j, 0].astype(bf16))
        b = b_ref[0].astype(f32)

        def conv_tile(j, t, r0):
            pieces = [slab_ref[j, t, pl.ds(r0 + ki, TR), kj:kj + W, :]
                      for ki in range(kh) for kj in range(kw)]
            patches = jnp.concatenate(pieces, axis=-1)
            acc = jax.lax.dot_general(
                patches, w_ref[...],
                dimension_numbers=(((2,), (0,)), ((), ())),
                preferred_element_type=f32)
            return acc + b

        def gate_math(j, acc, r0):
            th = jnp.tanh(acc)
            lane = jax.lax.broadcasted_iota(jnp.int32, acc.shape, 2)
            act = jnp.where(lane < 3 * hid, 0.5 * th + 0.5, th)
            i = act[..., 0 * hid:1 * hid]
            f = act[..., 1 * hid:2 * hid]
            o = act[..., 2 * hid:3 * hid]
            g = act[..., 3 * hid:4 * hid]
            c_cur = c_ref[j, pl.ds(r0, TR), :, :]
            c_next = f * c_cur + i * g
            c_ref[j, pl.ds(r0, TR), :, :] = c_next
            return o * jnp.tanh(c_next)

        def step(t, carry):
            for j in range(NB):
                slab_ref[j, t + 1, ph:ph + H, pw:pw + W, hid:hid + Cin] = (
                    x_ref[j, t + 1].astype(bf16))

            def row_tile(r, cc):
                r0 = pl.multiple_of(r * TR, TR)
                accs = [conv_tile(j, t, r0) for j in range(NB)]
                for j in range(NB):
                    h_next = gate_math(j, accs[j], r0)
                    slab_ref[j, t + 1, pl.ds(r0 + ph, TR), pw:pw + W,
                             0:hid] = h_next.astype(bf16)
                return cc
            jax.lax.fori_loop(0, NR, row_tile, 0, unroll=True)
            return carry

        jax.lax.fori_loop(0, T - 1, step, 0)

        def row_tile_last(r, cc):
            r0 = pl.multiple_of(r * TR, TR)
            accs = [conv_tile(j, T - 1, r0) for j in range(NB)]
            for j in range(NB):
                h_next = gate_math(j, accs[j], r0)
                out_ref[j, pl.ds(r0, TR), :, 0:hid] = (
                    h_next.astype(out_ref.dtype))
            return cc
        jax.lax.fori_loop(0, NR, row_tile_last, 0, unroll=True)
        for j in range(NB):
            out_ref[j, :, :, hid:2 * hid] = c_ref[j].astype(out_ref.dtype)

    return body


@functools.partial(jax.jit, static_argnames=("input_dim", "hidden_dim",
                                             "kernel_size"))
def _convlstm_seq(xs, hc0, w_packed, b_packed, *,
                  input_dim, hidden_dim, kernel_size):
    B, T, H, W, Cin = xs.shape
    hid = hidden_dim
    kh, kw = kernel_size
    C = Cin + hid
    Cpad = _round_up(C, 64)
    K = kh * kw * Cpad
    assert Cin == input_dim
    assert hc0.shape == (B, H, W, 2 * hid)
    assert w_packed.shape == (K, 4 * hid)

    TR = next((tr for tr in (8, 4, 2, 1) if H % tr == 0), 1)
    NB = 2 if B % 2 == 0 else 1

    ph, pw = kh // 2, kw // 2
    Hp, Wp = H + 2 * ph, W + 2 * pw

    gate_scale = jnp.concatenate([jnp.full((3 * hid,), 0.5, jnp.float32),
                                  jnp.ones((hid,), jnp.float32)])
    w_s = (w_packed * gate_scale).astype(jnp.bfloat16)
    b_s = b_packed * gate_scale

    body = _make_body(T, H, W, Cin, hid, kh, kw, Cpad, TR, NB)
    return pl.pallas_call(
        body,
        out_shape=jax.ShapeDtypeStruct((B, H, W, 2 * hid), xs.dtype),
        grid_spec=pltpu.PrefetchScalarGridSpec(
            num_scalar_prefetch=0,
            grid=(B // NB,),
            in_specs=[
                pl.BlockSpec((NB, T, H, W, Cin), lambda b: (b, 0, 0, 0, 0)),
                pl.BlockSpec((NB, H, W, 2 * hid), lambda b: (b, 0, 0, 0)),
                pl.BlockSpec((K, 4 * hid), lambda b: (0, 0)),
                pl.BlockSpec((1, 4 * hid), lambda b: (0, 0)),
            ],
            out_specs=pl.BlockSpec((NB, H, W, 2 * hid),
                                   lambda b: (b, 0, 0, 0)),
            scratch_shapes=[
                pltpu.VMEM((NB, T, Hp, Wp, Cpad), jnp.bfloat16),
                pltpu.VMEM((NB, H, W, hid), jnp.float32),
            ]),
        compiler_params=pltpu.CompilerParams(
            dimension_semantics=("parallel",)),
    )(xs, hc0, w_s, b_s)


def kernel(xs, hc0, w_packed, b_packed):
    return _convlstm_seq(xs, hc0, w_packed, b_packed,
                         input_dim=64, hidden_dim=64, kernel_size=(3, 3))


# speedup vs baseline: 1.6204x; 1.1419x over previous
import functools

import jax
import jax.numpy as jnp
from jax.experimental import pallas as pl
from jax.experimental.pallas import tpu as pltpu


def _round_up(v, m):
    return ((v + m - 1) // m) * m


def _make_body(T, H, W, Cin, hid, kh, kw, Cpad, TR, NB):
    ph, pw = kh // 2, kw // 2
    NR = H // TR
    Cx = Cpad - hid
    f32 = jnp.float32
    bf16 = jnp.bfloat16

    def body(x_ref, hc0_ref, w_ref, b_ref, out_ref, slab_ref, c_ref):
        def stage_x(j, t):
            xz = jnp.pad(x_ref[j, t].astype(bf16),
                         ((ph, ph), (pw, pw), (0, Cx - Cin)))
            slab_ref[j, t, :, :, hid:Cpad] = xz

        def store_h(j, t, r0, r, h2):
            top = ph if r == 0 else 0
            bot = ph if r == NR - 1 else 0
            hz = jnp.pad(h2.astype(bf16), ((top, bot), (pw, pw), (0, 0)))
            slab_ref[j, t, pl.ds(r0 + ph - top, TR + top + bot), :,
                     0:hid] = hz

        for j in range(NB):
            hc0 = hc0_ref[j].astype(f32)
            c_ref[j] = hc0[..., hid:2 * hid]
            stage_x(j, 0)
            h0z = jnp.pad((2.0 * hc0[..., :hid]).astype(bf16),
                          ((ph, ph), (pw, pw), (0, 0)))
            slab_ref[j, 0, :, :, 0:hid] = h0z
        b = b_ref[0].astype(f32)

        def conv_tile(j, t, r0):
            pieces = [slab_ref[j, t, pl.ds(r0 + ki, TR), kj:kj + W, :]
                      for ki in range(kh) for kj in range(kw)]
            patches = jnp.concatenate(pieces, axis=-1)
            acc = jax.lax.dot_general(
                patches, w_ref[...],
                dimension_numbers=(((2,), (0,)), ((), ())),
                preferred_element_type=f32)
            return acc + b

        def gate_math(j, acc, r0):
            th = jnp.tanh(acc)
            thi = th[..., 0 * hid:1 * hid]
            thf = th[..., 1 * hid:2 * hid]
            tho = th[..., 2 * hid:3 * hid]
            thg = th[..., 3 * hid:4 * hid]
            c_cur = c_ref[j, pl.ds(r0, TR), :, :]
            c_next = 0.5 * ((thf + 1.0) * c_cur + (thi + 1.0) * thg)
            c_ref[j, pl.ds(r0, TR), :, :] = c_next
            return (tho + 1.0) * jnp.tanh(c_next)

        def step(t, carry):
            for j in range(NB):
                stage_x(j, t + 1)

            for r in range(NR):
                r0 = r * TR
                for j in range(NB):
                    h2 = gate_math(j, conv_tile(j, t, r0), r0)
                    store_h(j, t + 1, r0, r, h2)
            return carry

        jax.lax.fori_loop(0, T - 1, step, 0)

        for r in range(NR):
            r0 = r * TR
            for j in range(NB):
                h2 = gate_math(j, conv_tile(j, T - 1, r0), r0)
                out_ref[j, pl.ds(r0, TR), :, 0:hid] = (
                    (0.5 * h2).astype(out_ref.dtype))
        for j in range(NB):
            out_ref[j, :, :, hid:2 * hid] = c_ref[j].astype(out_ref.dtype)

    return body


@functools.partial(jax.jit, static_argnames=("input_dim", "hidden_dim",
                                             "kernel_size"))
def _convlstm_seq(xs, hc0, w_packed, b_packed, *,
                  input_dim, hidden_dim, kernel_size):
    B, T, H, W, Cin = xs.shape
    hid = hidden_dim
    kh, kw = kernel_size
    C = Cin + hid
    Cpad = _round_up(C, 64)
    K = kh * kw * Cpad
    assert Cin == input_dim
    assert hc0.shape == (B, H, W, 2 * hid)
    assert w_packed.shape == (K, 4 * hid)

    TR = next((tr for tr in (8, 4, 2, 1) if H % tr == 0), 1)
    NB = 2 if B % 2 == 0 else 1

    ph, pw = kh // 2, kw // 2
    Hp, Wp = H + 2 * ph, W + 2 * pw

    gate_scale = jnp.concatenate([jnp.full((3 * hid,), 0.5, jnp.float32),
                                  jnp.ones((hid,), jnp.float32)])
    row_ids = jnp.arange(K) % Cpad
    row_scale = jnp.where(row_ids < hid, 0.5, 1.0).astype(jnp.float32)
    w_s = (w_packed * gate_scale * row_scale[:, None]).astype(jnp.bfloat16)
    b_s = b_packed * gate_scale

    body = _make_body(T, H, W, Cin, hid, kh, kw, Cpad, TR, NB)
    return pl.pallas_call(
        body,
        out_shape=jax.ShapeDtypeStruct((B, H, W, 2 * hid), xs.dtype),
        grid_spec=pltpu.PrefetchScalarGridSpec(
            num_scalar_prefetch=0,
            grid=(B // NB,),
            in_specs=[
                pl.BlockSpec((NB, T, H, W, Cin), lambda b: (b, 0, 0, 0, 0)),
                pl.BlockSpec((NB, H, W, 2 * hid), lambda b: (b, 0, 0, 0)),
                pl.BlockSpec((K, 4 * hid), lambda b: (0, 0)),
                pl.BlockSpec((1, 4 * hid), lambda b: (0, 0)),
            ],
            out_specs=pl.BlockSpec((NB, H, W, 2 * hid),
                                   lambda b: (b, 0, 0, 0)),
            scratch_shapes=[
                pltpu.VMEM((NB, T, Hp, Wp, Cpad), jnp.bfloat16),
                pltpu.VMEM((NB, H, W, hid), jnp.float32),
            ]),
        compiler_params=pltpu.CompilerParams(
            dimension_semantics=("parallel",)),
    )(xs, hc0, w_s, b_s)


def kernel(xs, hc0, w_packed, b_packed):
    return _convlstm_seq(xs, hc0, w_packed, b_packed,
                         input_dim=64, hidden_dim=64, kernel_size=(3, 3))
